# baseline (speedup 1.0000x reference)
import functools

import jax
import jax.numpy as jnp
from jax.experimental import pallas as pl

_K = 64


def _listalayer_block(x_ref, z_ref, w_ref, s_ref, o_ref):
    upd = jax.lax.dot_general(
        x_ref[...], w_ref[...], (((1,), (1,)), ((), ())),
        preferred_element_type=jnp.float32)
    upd = upd + jax.lax.dot_general(
        z_ref[...], s_ref[...], (((1,), (1,)), ((), ())),
        preferred_element_type=jnp.float32)
    bits = jax.lax.bitcast_convert_type(upd, jnp.int32) & jnp.int32(0x7FFFFFFF)
    rows, cols = upd.shape
    lo = jnp.zeros((rows, 1), jnp.int32)
    cnt_lo = jnp.full((rows, 1), cols, jnp.int32)
    hi = jnp.max(bits, axis=1, keepdims=True) + 1
    cnt_hi = jnp.zeros((rows, 1), jnp.int32)

    def _done(lo, hi, cnt_lo):
        return (cnt_lo == _K) | (hi <= lo + 1)

    def cond(state):
        lo, hi, cnt_lo, _, it = state
        return ~jnp.all(_done(lo, hi, cnt_lo)) & (it < 76)

    def body(state):
        lo, hi, cnt_lo, cnt_hi, it = state
        done = _done(lo, hi, cnt_lo)
        span = (hi - lo).astype(jnp.float32)
        frac = (cnt_lo - _K).astype(jnp.float32) / jnp.maximum(
            cnt_lo - cnt_hi, 1).astype(jnp.float32)
        mid = lo + (span * frac).astype(jnp.int32)
        mid = jnp.where(it < 40, mid, lo + ((hi - lo) >> 1))
        mid = jnp.clip(mid, lo + 1, jnp.maximum(hi - 1, lo + 1))
        cnt = jnp.sum((bits >= mid).astype(jnp.int32), axis=1, keepdims=True)
        ge = cnt >= _K
        act = ~done
        lo = jnp.where(act & ge, mid, lo)
        cnt_lo = jnp.where(act & ge, cnt, cnt_lo)
        hi = jnp.where(act & ~ge, mid, hi)
        cnt_hi = jnp.where(act & ~ge, cnt, cnt_hi)
        return lo, hi, cnt_lo, cnt_hi, it + 1

    lo, _, _, _, _ = jax.lax.while_loop(
        cond, body, (lo, hi, cnt_lo, cnt_hi, jnp.int32(0)))
    o_ref[...] = jnp.where(bits >= lo, upd, 0.0)


@functools.partial(jax.jit, static_argnames=("block_rows",))
def kernel(x, z_prev, W, S, block_rows: int = 256):
    batch, input_dim = x.shape
    code_dim = W.shape[0]
    grid = (batch // block_rows,)
    return pl.pallas_call(
        _listalayer_block,
        grid=grid,
        in_specs=[
            pl.BlockSpec((block_rows, input_dim), lambda i: (i, 0)),
            pl.BlockSpec((block_rows, code_dim), lambda i: (i, 0)),
            pl.BlockSpec((code_dim, input_dim), lambda i: (0, 0)),
            pl.BlockSpec((code_dim, code_dim), lambda i: (0, 0)),
        ],
        out_specs=pl.BlockSpec((block_rows, code_dim), lambda i: (i, 0)),
        out_shape=jax.ShapeDtypeStruct((batch, code_dim), jnp.float32),
    )(x, z_prev, W, S)


# speedup vs baseline: 1.7105x; 1.7105x over previous
import functools

import jax
import jax.numpy as jnp
from jax.experimental import pallas as pl
from jax.experimental.pallas import tpu as pltpu

_K = 64


def _matmul_into(x_ref, z_ref, w_ref, s_ref, buf):
    upd = jax.lax.dot_general(
        x_ref[...], w_ref[...], (((1,), (1,)), ((), ())),
        preferred_element_type=jnp.float32)
    upd = upd + jax.lax.dot_general(
        z_ref[...], s_ref[...], (((1,), (1,)), ((), ())),
        preferred_element_type=jnp.float32)
    buf[...] = upd


def _select_store(buf, o_ref):
    upd = buf[...]
    bits = jax.lax.bitcast_convert_type(upd, jnp.int32) & jnp.int32(0x7FFFFFFF)
    rows = upd.shape[0]
    t = jnp.zeros((rows, 1), jnp.int32)
    for b in range(30, -1, -1):
        cand = t | jnp.int32(1 << b)
        cnt = jnp.sum((bits >= cand).astype(jnp.int32), axis=1, keepdims=True)
        t = jnp.where(cnt >= _K, cand, t)
    o_ref[...] = jnp.where(bits >= t, upd, 0.0)


def _pipelined_block(x_ref, z_ref, w_ref, s_ref, o_ref, buf0, buf1, *,
                     nblocks):
    i = pl.program_id(0)

    @pl.when(i < nblocks)
    def _mm():
        @pl.when(i % 2 == 0)
        def _():
            _matmul_into(x_ref, z_ref, w_ref, s_ref, buf0)

        @pl.when(i % 2 == 1)
        def _():
            _matmul_into(x_ref, z_ref, w_ref, s_ref, buf1)

    @pl.when(i > 0)
    def _sel():
        @pl.when(i % 2 == 1)
        def _():
            _select_store(buf0, o_ref)

        @pl.when(i % 2 == 0)
        def _():
            _select_store(buf1, o_ref)


@functools.partial(jax.jit, static_argnames=("block_rows",))
def kernel(x, z_prev, W, S, block_rows: int = 256):
    batch, input_dim = x.shape
    code_dim = W.shape[0]
    nblocks = batch // block_rows
    grid = (nblocks + 1,)
    return pl.pallas_call(
        functools.partial(_pipelined_block, nblocks=nblocks),
        grid=grid,
        in_specs=[
            pl.BlockSpec((block_rows, input_dim),
                         lambda i: (jnp.minimum(i, nblocks - 1), 0)),
            pl.BlockSpec((block_rows, code_dim),
                         lambda i: (jnp.minimum(i, nblocks - 1), 0)),
            pl.BlockSpec((code_dim, input_dim), lambda i: (0, 0)),
            pl.BlockSpec((code_dim, code_dim), lambda i: (0, 0)),
        ],
        out_specs=pl.BlockSpec((block_rows, code_dim),
                               lambda i: (jnp.maximum(i - 1, 0), 0)),
        out_shape=jax.ShapeDtypeStruct((batch, code_dim), jnp.float32),
        scratch_shapes=[
            pltpu.VMEM((block_rows, code_dim), jnp.float32),
            pltpu.VMEM((block_rows, code_dim), jnp.float32),
        ],
    )(x, z_prev, W, S)


# speedup vs baseline: 1.7918x; 1.0475x over previous
import functools

import jax
import jax.numpy as jnp
from jax.experimental import pallas as pl
from jax.experimental.pallas import tpu as pltpu

_K = 64


def _matmul_into(x_ref, z_ref, w_ref, s_ref, buf):
    upd = jax.lax.dot_general(
        x_ref[...], w_ref[...], (((1,), (1,)), ((), ())),
        preferred_element_type=jnp.float32)
    upd = upd + jax.lax.dot_general(
        z_ref[...], s_ref[...], (((1,), (1,)), ((), ())),
        preferred_element_type=jnp.float32)
    buf[...] = upd


def _select_store(buf, o_ref):
    upd = buf[...]
    bits = jax.lax.bitcast_convert_type(upd, jnp.int32) & jnp.int32(0x7FFFFFFF)
    rows = upd.shape[0]
    t = jnp.zeros((rows, 1), jnp.int32)
    for b in range(30, -1, -1):
        cand = t | jnp.int32(1 << b)
        cnt = jnp.sum((bits >= cand).astype(jnp.int32), axis=1, keepdims=True)
        t = jnp.where(cnt >= _K, cand, t)
    o_ref[...] = jnp.where(bits >= t, upd, 0.0)


def _pipelined_block(x_ref, z_ref, w_ref, s_ref, o_ref, buf, *, nblocks):
    i = pl.program_id(0)
    _select_store(buf.at[(i + 1) % 2], o_ref)
    _matmul_into(x_ref, z_ref, w_ref, s_ref, buf.at[i % 2])


@functools.partial(jax.jit, static_argnames=("block_rows",))
def kernel(x, z_prev, W, S, block_rows: int = 256):
    batch, input_dim = x.shape
    code_dim = W.shape[0]
    nblocks = batch // block_rows
    grid = (nblocks + 1,)
    return pl.pallas_call(
        functools.partial(_pipelined_block, nblocks=nblocks),
        grid=grid,
        in_specs=[
            pl.BlockSpec((block_rows, input_dim),
                         lambda i: (jnp.minimum(i, nblocks - 1), 0)),
            pl.BlockSpec((block_rows, code_dim),
                         lambda i: (jnp.minimum(i, nblocks - 1), 0)),
            pl.BlockSpec((code_dim, input_dim), lambda i: (0, 0)),
            pl.BlockSpec((code_dim, code_dim), lambda i: (0, 0)),
        ],
        out_specs=pl.BlockSpec((block_rows, code_dim),
                               lambda i: (jnp.maximum(i - 1, 0), 0)),
        out_shape=jax.ShapeDtypeStruct((batch, code_dim), jnp.float32),
        scratch_shapes=[
            pltpu.VMEM((2, block_rows, code_dim), jnp.float32),
        ],
    )(x, z_prev, W, S)
